# baseline (speedup 1.0000x reference)
import functools

import jax
import jax.numpy as jnp
from jax import lax
from jax.experimental import pallas as pl
from jax.experimental.pallas import tpu as pltpu
from jax.experimental.pallas import tpu_sc as plsc

NC = 2
NS = 16
L = 16
NW = NC * NS
B = 16384
F = 128
BPW = B // NW
CHUNK = 128
NCHUNK = BPW // CHUNK

_mesh = plsc.VectorSubcoreMesh(
    core_axis_name="c", subcore_axis_name="s", num_cores=NC, num_subcores=NS
)


@functools.partial(
    pl.kernel,
    mesh=_mesh,
    out_type=jax.ShapeDtypeStruct((B,), jnp.float32),
    compiler_params=pltpu.CompilerParams(needs_layout_passes=False),
    scratch_types=[
        pltpu.VMEM((NCHUNK, CHUNK), jnp.int32),
        pltpu.VMEM((NCHUNK, CHUNK), jnp.int32),
        pltpu.VMEM((CHUNK, F), jnp.float32),
        pltpu.VMEM((CHUNK, F), jnp.float32),
        pltpu.VMEM((BPW,), jnp.float32),
        pltpu.SemaphoreType.DMA,
        pltpu.SemaphoreType.DMA,
    ],
)
def _mf_kernel(user_hbm, item_hbm, uf_hbm, if_hbm, out_hbm,
               uidx, iidx, urows, vrows, outv, semu, semv):
    wid = lax.axis_index("s") * NC + lax.axis_index("c")
    base = wid * BPW
    pltpu.sync_copy(user_hbm.at[wid], uidx)
    pltpu.sync_copy(item_hbm.at[wid], iidx)

    iota = lax.iota(jnp.int32, L)

    def do_chunk(c, carry):
        cu = pltpu.async_copy(uf_hbm.at[uidx.at[c]], urows, semu)
        cv = pltpu.async_copy(if_hbm.at[iidx.at[c]], vrows, semv)
        cu.wait()
        cv.wait()

        def do_group(g, carry2):
            base_r = g * L
            vec = jnp.zeros((L,), jnp.float32)
            for i in range(L):
                r = base_r + i
                acc = urows[r, pl.ds(0, L)] * vrows[r, pl.ds(0, L)]
                for j in range(1, F // L):
                    acc = acc + urows[r, pl.ds(j * L, L)] * vrows[r, pl.ds(j * L, L)]
                vec = jnp.where(iota == i, jnp.sum(acc), vec)
            outv[pl.ds(c * CHUNK + base_r, L)] = vec
            return carry2

        lax.fori_loop(0, CHUNK // L, do_group, 0)
        return carry

    lax.fori_loop(0, NCHUNK, do_chunk, 0)
    pltpu.sync_copy(outv, out_hbm.at[pl.ds(base, BPW)])


def kernel(user, item, user_factors, item_factors):
    u3 = user.reshape(NW, NCHUNK, CHUNK).astype(jnp.int32)
    i3 = item.reshape(NW, NCHUNK, CHUNK).astype(jnp.int32)
    return _mf_kernel(u3, i3, user_factors, item_factors)
